# Initial kernel scaffold; baseline (speedup 1.0000x reference)
import jax
import jax.numpy as jnp
from jax.experimental import pallas as pl


def kernel(embedding, memory_bank):
    raise NotImplementedError("write your pallas kernel here")


# speedup vs baseline: 2.0250x; 2.0250x over previous
import jax
import jax.numpy as jnp
from jax.experimental import pallas as pl
from jax.experimental.pallas import tpu as pltpu

Q = 3136
K = 16384
D = 1536
NN = 9

BQ = 784
RC = 112
BK = 1024
NQB = Q // BQ
NKB = K // BK
W = BK + 128


def _knn_body(emb_ref, memt_ref, out_ref, cand_ref, run_ref):
    k = pl.program_id(1)

    @pl.when(k == 0)
    def _init():
        run_ref[...] = jnp.full((BQ, 128), jnp.inf, dtype=jnp.float32)

    memt = memt_ref[...]
    m_sq = jnp.sum(memt * memt, axis=0, keepdims=True)
    d = jnp.dot(emb_ref[...], memt,
                preferred_element_type=jnp.float32)
    cand_ref[:, :BK] = m_sq - 2.0 * d
    cand_ref[:, BK:] = run_ref[...]

    def merge_chunk(c, carry):
        rows = pl.ds(c * RC, RC)
        for j in range(NN):
            cand = cand_ref[rows, :]
            iota = jax.lax.broadcasted_iota(jnp.int32, (RC, W), 1)
            mn = jnp.min(cand, axis=1, keepdims=True)
            pos = jnp.min(jnp.where(cand == mn, iota, W), axis=1,
                          keepdims=True)
            run_ref[rows, j:j + 1] = mn
            cand_ref[rows, :] = jnp.where(iota == pos, jnp.inf, cand)
        return carry

    jax.lax.fori_loop(0, BQ // RC, merge_chunk, 0)

    @pl.when(k == NKB - 1)
    def _finish():
        emb = emb_ref[...]
        q_sq = jnp.sum(emb * emb, axis=1, keepdims=True)
        sq = run_ref[...][:, :NN] + q_sq
        out_ref[...] = jnp.sqrt(jnp.maximum(sq, 1e-12))


@jax.jit
def kernel(embedding, memory_bank):
    memt = memory_bank.T
    return pl.pallas_call(
        _knn_body,
        grid=(NQB, NKB),
        in_specs=[
            pl.BlockSpec((BQ, D), lambda q, k: (q, 0)),
            pl.BlockSpec((D, BK), lambda q, k: (0, k)),
        ],
        out_specs=pl.BlockSpec((BQ, NN), lambda q, k: (q, 0)),
        out_shape=jax.ShapeDtypeStruct((Q, NN), jnp.float32),
        scratch_shapes=[
            pltpu.VMEM((BQ, W), jnp.float32),
            pltpu.VMEM((BQ, 128), jnp.float32),
        ],
        compiler_params=pltpu.CompilerParams(
            dimension_semantics=("arbitrary", "arbitrary")),
    )(embedding, memt)


# speedup vs baseline: 4.3455x; 2.1459x over previous
import jax
import jax.numpy as jnp
from jax.experimental import pallas as pl
from jax.experimental.pallas import tpu as pltpu

Q = 3136
K = 16384
D = 1536
NN = 9

BQ = 784
RC = 112
BK = 1024
NQB = Q // BQ
NKB = K // BK
NG = BK // 128 + 1

BIGF = 3.0e38
MASKHI = -2048

_NET = [(0, 1), (3, 4), (6, 7), (1, 2), (4, 5), (7, 8), (0, 1), (3, 4),
        (6, 7), (0, 3), (3, 6), (0, 3), (1, 4), (4, 7), (1, 4), (2, 5),
        (5, 8), (2, 5), (1, 3), (5, 7), (2, 6), (4, 6), (2, 4), (2, 3),
        (5, 6)]


def _knn_body(emb_ref, memt_ref, out_ref, cand_ref, run_ref):
    k = pl.program_id(1)

    @pl.when(k == 0)
    def _init():
        run_ref[...] = jnp.full((BQ, 128), BIGF, dtype=jnp.float32)

    memt = memt_ref[...]
    m_sq = jnp.sum(memt * memt, axis=0, keepdims=True)
    d = jnp.dot(emb_ref[...], memt,
                preferred_element_type=jnp.float32)
    sq = m_sq - 2.0 * d
    bits = jax.lax.bitcast_convert_type(sq, jnp.int32)
    col = jax.lax.broadcasted_iota(jnp.int32, (BQ, BK), 1)
    cand_ref[...] = jax.lax.bitcast_convert_type((bits & MASKHI) | col,
                                                 jnp.float32)

    def merge_chunk(c, carry):
        rows = pl.ds(c * RC, RC)
        rb = jax.lax.bitcast_convert_type(run_ref[rows, :], jnp.int32)
        rcol = BK + jax.lax.broadcasted_iota(jnp.int32, (RC, 128), 1)
        runk = jax.lax.bitcast_convert_type((rb & MASKHI) | rcol,
                                            jnp.float32)
        lvl = [cand_ref[rows, g * 128:(g + 1) * 128] for g in range(NG - 1)]
        lvl.append(runk)
        for i, j in _NET:
            lo = jnp.minimum(lvl[i], lvl[j])
            hi = jnp.maximum(lvl[i], lvl[j])
            lvl[i], lvl[j] = lo, hi
        outs = []
        for _ in range(NN):
            mn = jnp.min(lvl[0], axis=1, keepdims=True)
            won = lvl[0] == mn
            for g in range(NG - 1):
                lvl[g] = jnp.where(won, lvl[g + 1], lvl[g])
            lvl[NG - 1] = jnp.where(won, BIGF, lvl[NG - 1])
            outs.append(mn)
        outs.append(jnp.full((RC, 128 - NN), BIGF, dtype=jnp.float32))
        run_ref[rows, :] = jnp.concatenate(outs, axis=1)
        return carry

    jax.lax.fori_loop(0, BQ // RC, merge_chunk, 0)

    @pl.when(k == NKB - 1)
    def _finish():
        emb = emb_ref[...]
        q_sq = jnp.sum(emb * emb, axis=1, keepdims=True)
        keys = run_ref[...][:, :NN]
        vb = jax.lax.bitcast_convert_type(keys, jnp.int32) & MASKHI
        vals = jax.lax.bitcast_convert_type(vb, jnp.float32)
        out_ref[...] = jnp.sqrt(jnp.maximum(vals + q_sq, 1e-12))


@jax.jit
def kernel(embedding, memory_bank):
    memt = memory_bank.T
    return pl.pallas_call(
        _knn_body,
        grid=(NQB, NKB),
        in_specs=[
            pl.BlockSpec((BQ, D), lambda q, k: (q, 0)),
            pl.BlockSpec((D, BK), lambda q, k: (0, k)),
        ],
        out_specs=pl.BlockSpec((BQ, NN), lambda q, k: (q, 0)),
        out_shape=jax.ShapeDtypeStruct((Q, NN), jnp.float32),
        scratch_shapes=[
            pltpu.VMEM((BQ, BK), jnp.float32),
            pltpu.VMEM((BQ, 128), jnp.float32),
        ],
        compiler_params=pltpu.CompilerParams(
            dimension_semantics=("arbitrary", "arbitrary")),
    )(embedding, memt)
